# Initial kernel scaffold; baseline (speedup 1.0000x reference)
import jax
import jax.numpy as jnp
from jax.experimental import pallas as pl


def kernel(x, edge_index, edge_weight, W, b, ln_scale, ln_bias):
    raise NotImplementedError("write your pallas kernel here")


# speedup vs baseline: 30.7536x; 30.7536x over previous
import dataclasses
import functools

import jax
import jax.numpy as jnp
from jax import lax
from jax.experimental import pallas as pl
from jax.experimental.pallas import tpu as pltpu
from jax.experimental.pallas import tpu_sc as plsc

NC = 2
NS = 16
NW = NC * NS
LANES = 16
BLK = 128
D = 64
NBUF = 3


def _sc_mesh():
    return plsc.VectorSubcoreMesh(core_axis_name="c", subcore_axis_name="s")


def _sc_params():
    cp = pltpu.CompilerParams()
    fields = pltpu.CompilerParams.__dataclass_fields__
    if "needs_layout_passes" in fields:
        cp = dataclasses.replace(cp, needs_layout_passes=False)
    if "use_tc_tiling_on_sc" in fields:
        cp = dataclasses.replace(cp, use_tc_tiling_on_sc=False)
    return cp


def _make_deg_kernel(nb, npad):
    zc = npad // NS

    @functools.partial(
        pl.kernel,
        out_type=jax.ShapeDtypeStruct((NC, npad), jnp.float32),
        mesh=_sc_mesh(),
        scratch_types=[
            pltpu.VMEM((nb, BLK), jnp.int32),
            pltpu.VMEM((nb, BLK), jnp.float32),
            pltpu.VMEM((zc,), jnp.float32),
            pltpu.VMEM_SHARED((npad,), jnp.float32),
            pltpu.SemaphoreType.DMA,
        ],
        compiler_params=_sc_params(),
    )
    def deg_kernel(dst_hbm, w_hbm, out_hbm, dst_v, w_v, zb, deg_sh, sem):
        cid = lax.axis_index("c")
        sid = lax.axis_index("s")
        wid = sid * NC + cid

        @pl.loop(0, zc, step=LANES)
        def _(i):
            zb[pl.ds(i, LANES)] = jnp.zeros((LANES,), jnp.float32)

        pltpu.sync_copy(zb, deg_sh.at[pl.ds(sid * zc, zc)])
        plsc.subcore_barrier()

        pltpu.sync_copy(dst_hbm.at[wid], dst_v)
        pltpu.sync_copy(w_hbm.at[wid], w_v)

        descs = []
        for j in range(nb):
            descs.append(
                pltpu.async_copy(w_v.at[j], deg_sh.at[dst_v.at[j]], sem,
                                 add=True))
        for d in descs:
            d.wait()

        plsc.subcore_barrier()
        pltpu.sync_copy(deg_sh.at[pl.ds(sid * zc, zc)],
                        out_hbm.at[cid].at[pl.ds(sid * zc, zc)])

    return deg_kernel


def _make_agg_kernel(nb, npad):
    zrows = npad // NS

    @functools.partial(
        pl.kernel,
        out_type=jax.ShapeDtypeStruct((NC, npad, D), jnp.float32),
        mesh=_sc_mesh(),
        scratch_types=[
            pltpu.VMEM((nb, BLK), jnp.int32),
            pltpu.VMEM((nb, BLK), jnp.int32),
            pltpu.VMEM((nb, BLK), jnp.float32),
            pltpu.VMEM((NBUF, BLK, D), jnp.float32),
            pltpu.VMEM_SHARED((npad, D), jnp.float32),
            pltpu.SemaphoreType.DMA,
            pltpu.SemaphoreType.DMA,
        ],
        compiler_params=_sc_params(),
    )
    def agg_kernel(h2_hbm, src_hbm, dst_hbm, w_hbm, out_hbm,
                   src_v, dst_v, w_v, rows, s_sh, gsem, ssem):
        cid = lax.axis_index("c")
        sid = lax.axis_index("s")
        wid = sid * NC + cid

        @pl.loop(0, BLK)
        def _(i):
            for k in range(D // LANES):
                rows[0, i, pl.ds(k * LANES, LANES)] = jnp.zeros(
                    (LANES,), jnp.float32)

        for t in range(zrows // BLK):
            pltpu.sync_copy(rows.at[0],
                            s_sh.at[pl.ds(sid * zrows + t * BLK, BLK)])
        plsc.subcore_barrier()

        pltpu.sync_copy(src_hbm.at[wid], src_v)
        pltpu.sync_copy(dst_hbm.at[wid], dst_v)
        pltpu.sync_copy(w_hbm.at[wid], w_v)

        def gather(j):
            return pltpu.async_copy(h2_hbm.at[src_v.at[j]],
                                    rows.at[j % NBUF], gsem)

        def scale(j):
            b = j % NBUF

            @pl.loop(0, BLK)
            def _(i):
                wsp = plsc.load_gather(
                    w_v, [jnp.full((LANES,), j, jnp.int32),
                          jnp.full((LANES,), i, jnp.int32)])
                for k in range(D // LANES):
                    sl = pl.ds(k * LANES, LANES)
                    rows[b, i, sl] = rows[b, i, sl] * wsp

        def scatter(j):
            return pltpu.async_copy(rows.at[j % NBUF],
                                    s_sh.at[dst_v.at[j]], ssem, add=True)

        gd = {0: gather(0)}
        sd = {}
        for j in range(nb):
            if j >= 2:
                sd.pop(j - 2).wait()
            if j + 1 < nb:
                gd[j + 1] = gather(j + 1)
            gd.pop(j).wait()
            scale(j)
            sd[j] = scatter(j)
        for j in sorted(sd):
            sd.pop(j).wait()

        plsc.subcore_barrier()
        pltpu.sync_copy(s_sh.at[pl.ds(sid * zrows, zrows)],
                        out_hbm.at[cid].at[pl.ds(sid * zrows, zrows)])

    return agg_kernel


def _matmul_body(x_ref, w_ref, o_ref):
    o_ref[...] = jnp.dot(x_ref[...], w_ref[...],
                         preferred_element_type=jnp.float32,
                         precision=lax.Precision.HIGHEST)


def _h2_body(h_ref, degt_ref, o_ref):
    deg = degt_ref[:, 0:1] + degt_ref[:, 1:2] + 1.0
    dis = jnp.where(deg > 0, lax.rsqrt(jnp.where(deg > 0, deg, 1.0)), 0.0)
    o_ref[...] = h_ref[...] * dis


def _final_body(s2_ref, h2_ref, degt_ref, b_ref, lns_ref, lnb_ref, o_ref):
    deg = degt_ref[:, 0:1] + degt_ref[:, 1:2] + 1.0
    dis = jnp.where(deg > 0, lax.rsqrt(jnp.where(deg > 0, deg, 1.0)), 0.0)
    agg = dis * (s2_ref[0] + s2_ref[1] + h2_ref[...]) + b_ref[...]
    x1 = jnp.where(agg >= 0, agg, 0.01 * agg)
    mu = jnp.mean(x1, axis=-1, keepdims=True)
    var = jnp.mean((x1 - mu) ** 2, axis=-1, keepdims=True)
    x1 = (x1 - mu) * lax.rsqrt(var + 1e-5)
    o_ref[...] = x1 * lns_ref[...] + lnb_ref[...]


def kernel(x, edge_index, edge_weight, W, b, ln_scale, ln_bias):
    n, d_in = x.shape
    e = edge_weight.shape[0]
    nb = -(-e // (NW * BLK))
    epad = NW * nb * BLK
    npad = -(-n // (NS * BLK)) * (NS * BLK)

    src = jnp.pad(edge_index[0].astype(jnp.int32), (0, epad - e))
    dst = jnp.pad(edge_index[1].astype(jnp.int32), (0, epad - e))
    w = jnp.pad(edge_weight, (0, epad - e))
    src = src.reshape(NW, nb, BLK)
    dst = dst.reshape(NW, nb, BLK)
    w = w.reshape(NW, nb, BLK)

    deg2 = _make_deg_kernel(nb, npad)(dst, w)
    degt = deg2.T

    bm = 2000
    grid = (n // bm,)
    h = pl.pallas_call(
        _matmul_body,
        grid=grid,
        in_specs=[pl.BlockSpec((bm, d_in), lambda i: (i, 0)),
                  pl.BlockSpec((d_in, D), lambda i: (0, 0))],
        out_specs=pl.BlockSpec((bm, D), lambda i: (i, 0)),
        out_shape=jax.ShapeDtypeStruct((n, D), jnp.float32),
    )(x, W)

    h2 = pl.pallas_call(
        _h2_body,
        grid=grid,
        in_specs=[pl.BlockSpec((bm, D), lambda i: (i, 0)),
                  pl.BlockSpec((bm, 2), lambda i: (i, 0))],
        out_specs=pl.BlockSpec((bm, D), lambda i: (i, 0)),
        out_shape=jax.ShapeDtypeStruct((n, D), jnp.float32),
    )(h, degt)

    s2 = _make_agg_kernel(nb, npad)(h2, src, dst, w)

    out = pl.pallas_call(
        _final_body,
        grid=grid,
        in_specs=[pl.BlockSpec((NC, bm, D), lambda i: (0, i, 0)),
                  pl.BlockSpec((bm, D), lambda i: (i, 0)),
                  pl.BlockSpec((bm, 2), lambda i: (i, 0)),
                  pl.BlockSpec((1, D), lambda i: (0, 0)),
                  pl.BlockSpec((1, D), lambda i: (0, 0)),
                  pl.BlockSpec((1, D), lambda i: (0, 0))],
        out_specs=pl.BlockSpec((bm, D), lambda i: (i, 0)),
        out_shape=jax.ShapeDtypeStruct((n, D), jnp.float32),
    )(s2, h2, degt, b.reshape(1, D), ln_scale.reshape(1, D),
      ln_bias.reshape(1, D))
    return out
